# Initial kernel scaffold; baseline (speedup 1.0000x reference)
import functools

import jax
import jax.numpy as jnp
from jax import lax
from jax.experimental import pallas as pl
from jax.experimental.pallas import tpu as pltpu
from jax.experimental.pallas import tpu_sc as plsc

EMB_DIM = 32
N_TOKENS = 16384 * 50
NUM_WORKERS = 32
PER_WORKER = N_TOKENS // NUM_WORKERS
CHUNK = 128
NCHUNK = PER_WORKER // CHUNK

_mesh = plsc.VectorSubcoreMesh(core_axis_name="c", subcore_axis_name="s")


@functools.partial(
    pl.kernel,
    mesh=_mesh,
    out_type=jax.ShapeDtypeStruct((N_TOKENS, EMB_DIM), jnp.float32),
    scratch_types=[
        pltpu.VMEM((NCHUNK, CHUNK), jnp.int32),
        pltpu.VMEM((CHUNK, EMB_DIM), jnp.float32),
        pltpu.SemaphoreType.DMA,
    ],
)
def _sc_gather(idx_hbm, table_hbm, out_hbm, idx_v, rows_v, sem):
    wid = lax.axis_index("s") * 2 + lax.axis_index("c")
    pltpu.sync_copy(idx_hbm.at[wid], idx_v)
    base = wid * PER_WORKER

    def chunk_body(j, carry):
        pltpu.async_copy(table_hbm.at[idx_v.at[j]], rows_v, sem).wait()
        pltpu.sync_copy(rows_v, out_hbm.at[pl.ds(base + j * CHUNK, CHUNK)])
        return carry

    lax.fori_loop(0, NCHUNK, chunk_body, 0)


def kernel(token_ids, embedding_model):
    idx = token_ids.reshape(NUM_WORKERS, NCHUNK, CHUNK)
    out = _sc_gather(idx, embedding_model)
    return out.reshape(16384, 50, EMB_DIM)


# speedup vs baseline: 1.0229x; 1.0229x over previous
import functools

import jax
import jax.numpy as jnp
from jax import lax
from jax.experimental import pallas as pl
from jax.experimental.pallas import tpu as pltpu
from jax.experimental.pallas import tpu_sc as plsc

EMB_DIM = 32
N_TOKENS = 16384 * 50
NUM_WORKERS = 32
PER_WORKER = N_TOKENS // NUM_WORKERS
CHUNK = 128
NCHUNK = PER_WORKER // CHUNK

_mesh = plsc.VectorSubcoreMesh(core_axis_name="c", subcore_axis_name="s")


@functools.partial(
    pl.kernel,
    mesh=_mesh,
    out_type=jax.ShapeDtypeStruct((N_TOKENS, EMB_DIM), jnp.float32),
    scratch_types=[
        pltpu.VMEM((NCHUNK, CHUNK), jnp.int32),
        pltpu.VMEM((CHUNK, EMB_DIM), jnp.float32),
        pltpu.SemaphoreType.DMA,
    ],
    compiler_params=pltpu.CompilerParams(use_tc_tiling_on_sc=False),
)
def _sc_gather(idx_hbm, table_hbm, out_hbm, idx_v, rows_v, sem):
    wid = lax.axis_index("s") * 2 + lax.axis_index("c")
    pltpu.sync_copy(idx_hbm.at[wid], idx_v)
    base = wid * PER_WORKER

    def chunk_body(j, carry):
        pltpu.async_copy(table_hbm.at[idx_v.at[j]], rows_v, sem).wait()
        pltpu.sync_copy(rows_v, out_hbm.at[pl.ds(base + j * CHUNK, CHUNK)])
        return carry

    lax.fori_loop(0, NCHUNK, chunk_body, 0)


def kernel(token_ids, embedding_model):
    idx = token_ids.reshape(NUM_WORKERS, NCHUNK, CHUNK)
    out = _sc_gather(idx, embedding_model)
    return out.reshape(16384, 50, EMB_DIM)


# speedup vs baseline: 1.1138x; 1.0889x over previous
import functools

import jax
import jax.numpy as jnp
from jax import lax
from jax.experimental import pallas as pl
from jax.experimental.pallas import tpu as pltpu
from jax.experimental.pallas import tpu_sc as plsc

EMB_DIM = 32
N_TOKENS = 16384 * 50
NUM_WORKERS = 32
PER_WORKER = N_TOKENS // NUM_WORKERS
CHUNK = 128
NCHUNK = PER_WORKER // CHUNK
NB = 8

_mesh = plsc.VectorSubcoreMesh(core_axis_name="c", subcore_axis_name="s")


@functools.partial(
    pl.kernel,
    mesh=_mesh,
    out_type=jax.ShapeDtypeStruct((N_TOKENS, EMB_DIM), jnp.float32),
    scratch_types=[
        pltpu.VMEM((NCHUNK, CHUNK), jnp.int32),
        pltpu.VMEM((NB, CHUNK, EMB_DIM), jnp.float32),
    ] + [pltpu.SemaphoreType.DMA] * (2 * NB),
    compiler_params=pltpu.CompilerParams(use_tc_tiling_on_sc=False),
)
def _sc_gather(idx_hbm, table_hbm, out_hbm, idx_v, rows_v, *sems):
    gat_sems, out_sems = sems[:NB], sems[NB:]
    wid = lax.axis_index("s") * 2 + lax.axis_index("c")
    pltpu.sync_copy(idx_hbm.at[wid], idx_v)
    base = wid * PER_WORKER

    def gat_copy(c, b):
        return pltpu.make_async_copy(
            table_hbm.at[idx_v.at[c]], rows_v.at[b], gat_sems[b])

    def out_copy(c, b):
        return pltpu.make_async_copy(
            rows_v.at[b], out_hbm.at[pl.ds(base + c * CHUNK, CHUNK)],
            out_sems[b])

    for b in range(NB - 1):
        gat_copy(b, b).start()

    def outer(c0, carry):
        for b in range(NB):
            c = c0 * NB + b
            bp = (b - 1) % NB
            gat_copy(c, b).wait()
            out_copy(c, b).start()

            @pl.when(c >= 1)
            def _():
                out_copy(c - 1, bp).wait()

            @pl.when(c + NB - 1 < NCHUNK)
            def _():
                gat_copy(c + NB - 1, bp).start()
        return carry

    lax.fori_loop(0, NCHUNK // NB, outer, 0)
    out_copy(NCHUNK - 1, (NCHUNK - 1) % NB).wait()


def kernel(token_ids, embedding_model):
    idx = token_ids.reshape(NUM_WORKERS, NCHUNK, CHUNK)
    out = _sc_gather(idx, embedding_model)
    return out.reshape(16384, 50, EMB_DIM)


# speedup vs baseline: 1.2026x; 1.0797x over previous
import functools

import jax
import jax.numpy as jnp
from jax import lax
from jax.experimental import pallas as pl
from jax.experimental.pallas import tpu as pltpu
from jax.experimental.pallas import tpu_sc as plsc

NUM_EMB = 1_000_000
EMB_DIM = 32
N_I = 16384
N_J = 50
NUM_WORKERS = 32
N_BLOCKS = N_J * (N_I // 128)
BLOCKS_PER_W = N_BLOCKS // NUM_WORKERS

_mesh = plsc.VectorSubcoreMesh(core_axis_name="c", subcore_axis_name="s")


@functools.partial(
    pl.kernel,
    mesh=_mesh,
    out_type=jax.ShapeDtypeStruct((N_J, EMB_DIM, N_I), jnp.float32),
    scratch_types=[
        pltpu.VMEM((128,), jnp.int32),
        pltpu.VMEM((128,), jnp.int32),
        pltpu.VMEM((128, 128), jnp.float32),
        pltpu.VMEM((EMB_DIM, 128), jnp.float32),
        pltpu.SemaphoreType.DMA,
    ],
    compiler_params=pltpu.CompilerParams(needs_layout_passes=False),
)
def _sc_gather(tok_ref, rp_ref, out_ref, idx_v, n_v, rows_v, blk_v, sem):
    wid = lax.axis_index("s") * 2 + lax.axis_index("c")
    iota = lax.iota(jnp.int32, 16)

    def block_body(k, carry):
        b = wid * BLOCKS_PER_W + k
        j = b // 128
        tc = b % 128
        pltpu.sync_copy(tok_ref.at[j, pl.ds(tc * 128, 128)], idx_v)

        def build(h, c2):
            v = idx_v[pl.ds(h * 16, 16)]
            n_v[pl.ds(h * 16, 16)] = lax.shift_right_logical(v, 2)
            return c2

        lax.fori_loop(0, 8, build, 0)
        pltpu.async_copy(rp_ref.at[n_v], rows_v, sem).wait()

        def extract(h, c2):
            vo = idx_v[pl.ds(h * 16, 16)]
            col = (vo & 3) * 32
            rowi = iota + h * 16
            for d in range(EMB_DIM):
                blk_v[d, pl.ds(h * 16, 16)] = plsc.load_gather(
                    rows_v, [rowi, col + d])
            return c2

        lax.fori_loop(0, 8, extract, 0)
        pltpu.sync_copy(blk_v, out_ref.at[j, :, pl.ds(tc * 128, 128)])
        return carry

    lax.fori_loop(0, BLOCKS_PER_W, block_body, 0)


def kernel(token_ids, embedding_model):
    tok_t = token_ids.T
    rp = jnp.reshape(embedding_model, (NUM_EMB // 4, 128))
    out_t = _sc_gather(tok_t, rp)
    return jnp.transpose(out_t, (2, 0, 1))


# speedup vs baseline: 1.6310x; 1.3562x over previous
import functools

import jax
import jax.numpy as jnp
from jax import lax
from jax.experimental import pallas as pl
from jax.experimental.pallas import tpu as pltpu
from jax.experimental.pallas import tpu_sc as plsc

NUM_EMB = 1_000_000
EMB_DIM = 32
N_I = 16384
N_J = 50
NUM_WORKERS = 32
N_BLOCKS = N_J * (N_I // 128)
BLOCKS_PER_W = N_BLOCKS // NUM_WORKERS

_mesh = plsc.VectorSubcoreMesh(core_axis_name="c", subcore_axis_name="s")


@functools.partial(
    pl.kernel,
    mesh=_mesh,
    out_type=jax.ShapeDtypeStruct((N_J, EMB_DIM, N_I), jnp.float32),
    scratch_types=[
        pltpu.VMEM((8, 128), jnp.int32),
        pltpu.VMEM((8, 128), jnp.int32),
        pltpu.VMEM((4, 128, 128), jnp.float32),
        pltpu.VMEM((4, EMB_DIM, 128), jnp.float32),
    ] + [pltpu.SemaphoreType.DMA] * 16,
    compiler_params=pltpu.CompilerParams(needs_layout_passes=False),
)
def _sc_gather(tok_ref, rp_ref, out_ref, idx_v, n_v, rows_v, blk_v, *sems):
    idx_sems, gat_sems, out_sems = sems[:8], sems[8:12], sems[12:16]
    wid = lax.axis_index("s") * 2 + lax.axis_index("c")
    iota = lax.iota(jnp.int32, 16)

    def idx_copy(c, s):
        b = wid * BLOCKS_PER_W + c
        return pltpu.make_async_copy(
            tok_ref.at[b // 128, pl.ds((b % 128) * 128, 128)],
            idx_v.at[s], idx_sems[s])

    def gat_copy(c, s, r):
        return pltpu.make_async_copy(
            rp_ref.at[n_v.at[s]], rows_v.at[r], gat_sems[r])

    def out_copy(c, r):
        b = wid * BLOCKS_PER_W + c
        return pltpu.make_async_copy(
            blk_v.at[r], out_ref.at[b // 128, :, pl.ds((b % 128) * 128, 128)],
            out_sems[r])

    def build_n(s):
        def h_body(h, c2):
            v = idx_v[s, pl.ds(h * 16, 16)]
            n_v[s, pl.ds(h * 16, 16)] = lax.shift_right_logical(v, 2)
            return c2
        lax.fori_loop(0, 8, h_body, 0)

    def extract(s, r):
        def h_body(h, c2):
            vo = idx_v[s, pl.ds(h * 16, 16)]
            col = (vo & 3) * 32
            rowi = iota + h * 16
            for d in range(EMB_DIM):
                blk_v[r, d, pl.ds(h * 16, 16)] = plsc.load_gather(
                    rows_v.at[r], [rowi, col + d])
            return c2
        lax.fori_loop(0, 8, h_body, 0)

    for c in range(5):
        idx_copy(c, c % 8).start()
    for c in range(3):
        idx_copy(c, c % 8).wait()
        build_n(c % 8)
        gat_copy(c, c % 8, c % 4).start()

    def outer(c0, carry):
        for u in range(8):
            c = c0 * 8 + u
            s, r = u, u % 4
            gat_copy(c, s, r).wait()

            @pl.when(c >= 4)
            def _():
                out_copy(c - 4, r).wait()

            extract(s, r)
            out_copy(c, r).start()

            @pl.when(c + 5 < BLOCKS_PER_W)
            def _():
                idx_copy(c + 5, (u + 5) % 8).start()

            @pl.when(c + 3 < BLOCKS_PER_W)
            def _():
                idx_copy(c + 3, (u + 3) % 8).wait()
                build_n((u + 3) % 8)
                gat_copy(c + 3, (u + 3) % 8, (u + 3) % 4).start()
        return carry

    lax.fori_loop(0, BLOCKS_PER_W // 8, outer, 0)
    for c in range(BLOCKS_PER_W - 4, BLOCKS_PER_W):
        out_copy(c, c % 4).wait()


def kernel(token_ids, embedding_model):
    tok_t = token_ids.T
    rp = jnp.reshape(embedding_model, (NUM_EMB // 4, 128))
    out_t = _sc_gather(tok_t, rp)
    return jnp.transpose(out_t, (2, 0, 1))
